# baseline (speedup 1.0000x reference)
import functools

import jax
import jax.numpy as jnp
from jax import lax
from jax.experimental import pallas as pl
from jax.experimental.pallas import tpu as pltpu
from jax.experimental.pallas import tpu_sc as plsc

BATCH = 16384
NUM_VERBS = 117
NUM_HOIS = 600

NC = 2
NS = 16
NW = NC * NS
COLS_PER_TILE = BATCH // NW
CCHUNK = 128
NCHUNK = COLS_PER_TILE // CCHUNK
ROWS_H0 = 200
NPARTS = NUM_HOIS // ROWS_H0

_mesh = plsc.VectorSubcoreMesh(
    core_axis_name="c", subcore_axis_name="s", num_cores=NC, num_subcores=NS
)


@functools.partial(
    pl.kernel,
    out_type=jax.ShapeDtypeStruct((NUM_HOIS, BATCH), jnp.float32),
    mesh=_mesh,
    scratch_types=[
        pltpu.VMEM((NUM_HOIS + 16,), jnp.int32),
        pltpu.VMEM((NUM_VERBS, CCHUNK), jnp.float32),
        pltpu.VMEM((NUM_VERBS, CCHUNK), jnp.float32),
        pltpu.VMEM((ROWS_H0, CCHUNK), jnp.float32),
        pltpu.VMEM((ROWS_H0, CCHUNK), jnp.float32),
        pltpu.VMEM((ROWS_H0, CCHUNK), jnp.float32),
        pltpu.SemaphoreType.DMA,
        pltpu.SemaphoreType.DMA,
        pltpu.SemaphoreType.DMA,
        pltpu.SemaphoreType.DMA,
        pltpu.SemaphoreType.DMA,
        pltpu.SemaphoreType.DMA,
    ],
    compiler_params=pltpu.CompilerParams(
        needs_layout_passes=False, skip_device_barrier=True
    ),
)
def _scatter_verbs_kernel(
    vt_hbm, idx_hbm, out_hbm, idx_v, in0, in1, ob0, ob1, ob2,
    sx, si0, si1, so0, so1, so2,
):
    wid = lax.axis_index("s") * NC + lax.axis_index("c")
    base_col = wid * COLS_PER_TILE

    in_bufs, sin = [in0, in1], [si0, si1]
    obufs, sout = [ob0, ob1, ob2], [so0, so1, so2]
    halves = [(p * ROWS_H0, ROWS_H0) for p in range(NPARTS)]

    def start_in(c):
        col0 = base_col + c * CCHUNK
        return pltpu.async_copy(
            vt_hbm.at[:, pl.ds(col0, CCHUNK)], in_bufs[c & 1], sin[c & 1]
        )

    hidx = pltpu.async_copy(idx_hbm, idx_v.at[pl.ds(0, NUM_HOIS)], sx)
    hin = {0: start_in(0)}
    hpad = pltpu.async_copy(
        idx_hbm.at[pl.ds(0, 16)], idx_v.at[pl.ds(NUM_HOIS, 16)], sx
    )
    hidx.wait()
    hpad.wait()

    def compute_half(in_v, p):
        h0, nh = halves[p]
        out_v = obufs[p]

        @plsc.parallel_loop(0, nh, step=1, unroll=4)
        def row_body(r):
            v = idx_v[pl.ds(h0 + r, 16)][0]
            for k in range(CCHUNK // 16):
                out_v[r, pl.ds(16 * k, 16)] = in_v[v, pl.ds(16 * k, 16)]

    def start_out(c, p):
        col0 = base_col + c * CCHUNK
        h0, nh = halves[p]
        return pltpu.async_copy(
            obufs[p].at[pl.ds(0, nh)],
            out_hbm.at[pl.ds(h0, nh), pl.ds(col0, CCHUNK)],
            sout[p],
        )

    hout = {}
    for c in range(NCHUNK):
        if c + 1 < NCHUNK:
            hin[c + 1] = start_in(c + 1)
        hin[c].wait()
        for p in range(NPARTS):
            if c > 0:
                hout[(c - 1, p)].wait()
            compute_half(in_bufs[c & 1], p)
            hout[(c, p)] = start_out(c, p)
    for p in range(NPARTS):
        hout[(NCHUNK - 1, p)].wait()


def kernel(verb_scores, hoi_verb_map):
    hmap = hoi_verb_map.astype(jnp.int32)
    out_t = _scatter_verbs_kernel(verb_scores.T, hmap)
    return out_t.T


# speedup vs baseline: 1.0209x; 1.0209x over previous
import functools

import jax
import jax.numpy as jnp
from jax import lax
from jax.experimental import pallas as pl
from jax.experimental.pallas import tpu as pltpu
from jax.experimental.pallas import tpu_sc as plsc

BATCH = 16384
NUM_VERBS = 117
NUM_HOIS = 600

NC = 2
NS = 16
NW = NC * NS
COLS_PER_TILE = BATCH // NW
CCHUNK = 128
NCHUNK = COLS_PER_TILE // CCHUNK
ROWS_H0 = 304
NPARTS = 2

_mesh = plsc.VectorSubcoreMesh(
    core_axis_name="c", subcore_axis_name="s", num_cores=NC, num_subcores=NS
)


@functools.partial(
    pl.kernel,
    out_type=jax.ShapeDtypeStruct((NUM_HOIS, BATCH), jnp.float32),
    mesh=_mesh,
    scratch_types=[
        pltpu.VMEM((NUM_HOIS + 16,), jnp.int32),
        pltpu.VMEM((NUM_VERBS, CCHUNK), jnp.float32),
        pltpu.VMEM((NUM_VERBS, CCHUNK), jnp.float32),
        pltpu.VMEM((ROWS_H0, CCHUNK), jnp.float32),
        pltpu.VMEM((ROWS_H0, CCHUNK), jnp.float32),
        pltpu.SemaphoreType.DMA,
        pltpu.SemaphoreType.DMA,
        pltpu.SemaphoreType.DMA,
        pltpu.SemaphoreType.DMA,
        pltpu.SemaphoreType.DMA,
    ],
    compiler_params=pltpu.CompilerParams(
        needs_layout_passes=False, skip_device_barrier=True
    ),
)
def _scatter_verbs_kernel(
    vt_hbm, idx_hbm, out_hbm, idx_v, in0, in1, ob0, ob1, sx, si0, si1, so0, so1
):
    wid = lax.axis_index("s") * NC + lax.axis_index("c")
    base_col = wid * COLS_PER_TILE

    in_bufs, sin = [in0, in1], [si0, si1]
    obufs, sout = [ob0, ob1], [so0, so1]
    halves = [(0, ROWS_H0), (ROWS_H0, NUM_HOIS - ROWS_H0)]

    def start_in(c):
        col0 = base_col + c * CCHUNK
        return pltpu.async_copy(
            vt_hbm.at[:, pl.ds(col0, CCHUNK)], in_bufs[c & 1], sin[c & 1]
        )

    hidx = pltpu.async_copy(idx_hbm, idx_v.at[pl.ds(0, NUM_HOIS)], sx)
    hin = {0: start_in(0)}
    hpad = pltpu.async_copy(
        idx_hbm.at[pl.ds(0, 16)], idx_v.at[pl.ds(NUM_HOIS, 16)], sx
    )
    hidx.wait()
    hpad.wait()

    def compute_half(in_v, p):
        h0, nh = halves[p]
        out_v = obufs[p]

        @plsc.parallel_loop(0, nh, step=1, unroll=2)
        def row_body(r):
            v = idx_v[pl.ds(h0 + r, 16)][0]
            for k in range(CCHUNK // 16):
                out_v[r, pl.ds(16 * k, 16)] = in_v[v, pl.ds(16 * k, 16)]

    def start_out(c, p):
        col0 = base_col + c * CCHUNK
        h0, nh = halves[p]
        return pltpu.async_copy(
            obufs[p].at[pl.ds(0, nh)],
            out_hbm.at[pl.ds(h0, nh), pl.ds(col0, CCHUNK)],
            sout[p],
        )

    hout = {}
    for c in range(NCHUNK):
        if c + 1 < NCHUNK:
            hin[c + 1] = start_in(c + 1)
        hin[c].wait()
        for p in range(NPARTS):
            if c > 0:
                hout[(c - 1, p)].wait()
            compute_half(in_bufs[c & 1], p)
            hout[(c, p)] = start_out(c, p)
    for p in range(NPARTS):
        hout[(NCHUNK - 1, p)].wait()


def kernel(verb_scores, hoi_verb_map):
    hmap = hoi_verb_map.astype(jnp.int32)
    out_t = _scatter_verbs_kernel(verb_scores.T, hmap)
    return out_t.T
